# baseline (speedup 1.0000x reference)
import functools

import jax
import jax.numpy as jnp
from jax import lax
from jax.experimental import pallas as pl
from jax.experimental.pallas import tpu as pltpu
from jax.experimental.pallas import tpu_sc as plsc

T = 32768
V = 100000
D = 64
U = 16
DP = 128

_NC = 2
_NS = 16
_NW = _NC * _NS
_ROWS_PER_W = T // _NW
_CHUNK = 128
_NCHUNK = _ROWS_PER_W // _CHUNK


def _gather_body(table_hbm, idx_hbm, out_hbm, idx_v, rows_v, sem):
    wid = lax.axis_index("s") * _NC + lax.axis_index("c")
    pltpu.sync_copy(idx_hbm.at[wid], idx_v)
    base = wid * _ROWS_PER_W
    for j in range(_NCHUNK):
        pltpu.async_copy(table_hbm.at[idx_v.at[j]], rows_v, sem).wait()
        pltpu.sync_copy(rows_v, out_hbm.at[pl.ds(base + j * _CHUNK, _CHUNK)])


def _sc_gather(table, idx):
    mesh = plsc.VectorSubcoreMesh(core_axis_name="c", subcore_axis_name="s")
    k = functools.partial(
        pl.kernel,
        mesh=mesh,
        out_type=jax.ShapeDtypeStruct((T, DP), jnp.float32),
        scratch_types=[
            pltpu.VMEM((_NCHUNK, _CHUNK), jnp.int32),
            pltpu.VMEM((_CHUNK, DP), jnp.float32),
            pltpu.SemaphoreType.DMA,
        ],
        compiler_params=pltpu.CompilerParams(use_tc_tiling_on_sc=True),
    )(_gather_body)
    return k(table, idx)


_BC = 4096


def _pad_body(tin_ref, out_ref):
    eye = jnp.eye(D, dtype=jnp.float32)
    t = lax.dot_general(tin_ref[...], eye, (((0,), (0,)), ((), ())),
                        preferred_element_type=jnp.float32)
    out_ref[...] = jnp.concatenate(
        [t, jnp.zeros((_BC, DP - D), jnp.float32)], axis=1)


def _pad_table(tbl_t):
    return pl.pallas_call(
        _pad_body,
        grid=(pl.cdiv(V, _BC),),
        in_specs=[pl.BlockSpec((D, _BC), lambda i: (0, i))],
        out_specs=pl.BlockSpec((_BC, DP), lambda i: (i, 0)),
        out_shape=jax.ShapeDtypeStruct((V, DP), jnp.float32),
    )(tbl_t)


_BT = 1024
_GRID = T // _BT


def _morph_body(bnd_ref, emb_ref, m_ref, out_ref):
    g = pl.program_id(0)
    v = emb_ref[...]
    vb = v.astype(jnp.bfloat16)
    tok = g * _BT + lax.broadcasted_iota(jnp.int32, (_BT, 1), 0)
    zero = jnp.zeros((), jnp.bfloat16)
    parts = []
    for u in range(U):
        m_u = jnp.logical_and(tok >= bnd_ref[u], tok < bnd_ref[u + 1])
        parts.append(jnp.where(m_u, vb, zero))
    b = jnp.concatenate(parts, axis=1)
    morph = lax.dot_general(b, m_ref[...], (((1,), (0,)), ((), ())),
                            preferred_element_type=jnp.float32)
    acc = v[:, :D] + morph
    n = jnp.sqrt(jnp.sum(acc * acc, axis=1, keepdims=True))
    out_ref[...] = acc / jnp.maximum(n, 1e-12)


def _tc_morph(bnd, emb128, m_flat, interpret=False):
    grid_spec = pltpu.PrefetchScalarGridSpec(
        num_scalar_prefetch=1,
        grid=(_GRID,),
        in_specs=[
            pl.BlockSpec((_BT, DP), lambda i, bnd: (i, 0)),
            pl.BlockSpec((U * DP, D), lambda i, bnd: (0, 0)),
        ],
        out_specs=pl.BlockSpec((_BT, D), lambda i, bnd: (i, 0)),
    )
    return pl.pallas_call(
        _morph_body,
        grid_spec=grid_spec,
        out_shape=jax.ShapeDtypeStruct((T, D), jnp.float32),
        interpret=interpret,
    )(bnd, emb128, m_flat)


def kernel(novel_items, novel_userids, item_emb, seg2_out):
    idx = novel_items.astype(jnp.int32).reshape(_NW, _NCHUNK, _CHUNK)
    tbl = _pad_table(jnp.transpose(item_emb))
    emb128 = _sc_gather(tbl, idx)
    m_flat = jnp.pad(seg2_out, ((0, 0), (0, DP - D), (0, 0))
                     ).reshape(U * DP, D).astype(jnp.bfloat16)
    bnd = jnp.searchsorted(
        novel_userids, jnp.arange(U + 1, dtype=novel_userids.dtype)
    ).astype(jnp.int32)
    return _tc_morph(bnd, emb128, m_flat)


# speedup vs baseline: 1.5175x; 1.5175x over previous
import functools

import jax
import jax.numpy as jnp
from jax import lax
from jax.experimental import pallas as pl
from jax.experimental.pallas import tpu as pltpu
from jax.experimental.pallas import tpu_sc as plsc

T = 32768
V = 100000
D = 64
U = 16
DP = 128

_NC = 2
_NS = 16
_NW = _NC * _NS
_ROWS_PER_W = T // _NW
_CHUNK = 128
_NCHUNK = _ROWS_PER_W // _CHUNK


def _gather_body(table_hbm, idx_hbm, out_hbm, idx_v, rows_v, sem):
    wid = lax.axis_index("s") * _NC + lax.axis_index("c")
    pltpu.sync_copy(idx_hbm.at[wid], idx_v)
    base = wid * _ROWS_PER_W
    for j in range(_NCHUNK):
        pltpu.async_copy(table_hbm.at[idx_v.at[j]], rows_v, sem).wait()
        pltpu.sync_copy(rows_v, out_hbm.at[pl.ds(base + j * _CHUNK, _CHUNK)])


def _sc_gather(table, idx):
    mesh = plsc.VectorSubcoreMesh(core_axis_name="c", subcore_axis_name="s")
    k = functools.partial(
        pl.kernel,
        mesh=mesh,
        out_type=jax.ShapeDtypeStruct((T, DP), jnp.float32),
        scratch_types=[
            pltpu.VMEM((_NCHUNK, _CHUNK), jnp.int32),
            pltpu.VMEM((_CHUNK, DP), jnp.float32),
            pltpu.SemaphoreType.DMA,
        ],
        compiler_params=pltpu.CompilerParams(use_tc_tiling_on_sc=True),
    )(_gather_body)
    return k(table, idx)


_BC = 4096


def _pad_body(tin_ref, out_ref):
    eye = jnp.eye(D, dtype=jnp.float32)
    t = lax.dot_general(tin_ref[...], eye, (((0,), (0,)), ((), ())),
                        preferred_element_type=jnp.float32)
    out_ref[...] = jnp.concatenate([t, t], axis=1)


def _pad_table(tbl_t):
    return pl.pallas_call(
        _pad_body,
        grid=(pl.cdiv(V, _BC),),
        in_specs=[pl.BlockSpec((D, _BC), lambda i: (0, i))],
        out_specs=pl.BlockSpec((_BC, DP), lambda i: (i, 0)),
        out_shape=jax.ShapeDtypeStruct((V, DP), jnp.float32),
    )(tbl_t)


_BT = 1024
_GRID = T // _BT


def _morph_body(bnd_ref, emb_ref, m_ref, out_ref):
    g = pl.program_id(0)
    v = emb_ref[...]
    eye = jnp.eye(DP, dtype=jnp.float32)
    vt = lax.dot_general(eye, v, (((1,), (1,)), ((), ())),
                         preferred_element_type=jnp.float32)
    e_t = vt[:D, :]
    e_bf = e_t.astype(jnp.bfloat16)
    tok = g * _BT + lax.broadcasted_iota(jnp.int32, (1, _BT), 1)
    parts = []
    for u in range(U):
        m_u = jnp.logical_and(tok >= bnd_ref[u], tok < bnd_ref[u + 1])
        parts.append(e_bf * m_u.astype(jnp.bfloat16))
    b_t = jnp.concatenate(parts, axis=0)
    morph_t = lax.dot_general(m_ref[...], b_t, (((1,), (0,)), ((), ())),
                              preferred_element_type=jnp.float32)
    acc = e_t + morph_t
    n = jnp.sqrt(jnp.sum(acc * acc, axis=0, keepdims=True))
    out_ref[...] = acc / jnp.maximum(n, 1e-12)


def _tc_morph(bnd, emb128, m_t, interpret=False):
    grid_spec = pltpu.PrefetchScalarGridSpec(
        num_scalar_prefetch=1,
        grid=(_GRID,),
        in_specs=[
            pl.BlockSpec((_BT, DP), lambda i, bnd: (i, 0)),
            pl.BlockSpec((D, U * D), lambda i, bnd: (0, 0)),
        ],
        out_specs=pl.BlockSpec((D, _BT), lambda i, bnd: (0, i)),
    )
    return pl.pallas_call(
        _morph_body,
        grid_spec=grid_spec,
        out_shape=jax.ShapeDtypeStruct((D, T), jnp.float32),
        interpret=interpret,
    )(bnd, emb128, m_t)


def kernel(novel_items, novel_userids, item_emb, seg2_out):
    idx = novel_items.astype(jnp.int32).reshape(_NW, _NCHUNK, _CHUNK)
    tbl = _pad_table(jnp.transpose(item_emb))
    emb128 = _sc_gather(tbl, idx)
    m_t = jnp.transpose(seg2_out, (2, 0, 1)).reshape(D, U * D).astype(jnp.bfloat16)
    bnd = jnp.searchsorted(
        novel_userids, jnp.arange(U + 1, dtype=novel_userids.dtype)
    ).astype(jnp.int32)
    return jnp.transpose(_tc_morph(bnd, emb128, m_t))


# speedup vs baseline: 1.7852x; 1.1764x over previous
import functools

import jax
import jax.numpy as jnp
from jax import lax
from jax.experimental import pallas as pl
from jax.experimental.pallas import tpu as pltpu
from jax.experimental.pallas import tpu_sc as plsc

T = 32768
V = 100000
D = 64
U = 16
DP = 128

_NC = 2
_NS = 16
_NW = _NC * _NS
_ROWS_PER_W = T // _NW
_CHUNK = 128
_NCHUNK = _ROWS_PER_W // _CHUNK


def _gather_body(table_hbm, idx_hbm, out_hbm, idx_v, rows_a, rows_b,
                 gs0, gs1, os0, os1):
    wid = lax.axis_index("s") * _NC + lax.axis_index("c")
    pltpu.sync_copy(idx_hbm.at[wid], idx_v)
    base = wid * _ROWS_PER_W
    bufs, gsems, osems = (rows_a, rows_b), (gs0, gs1), (os0, os1)
    descs_g = [None] * _NCHUNK
    descs_o = [None] * _NCHUNK
    descs_g[0] = pltpu.async_copy(table_hbm.at[idx_v.at[0]], bufs[0], gsems[0])
    for j in range(_NCHUNK):
        b = j % 2
        if j + 1 < _NCHUNK:
            nb = (j + 1) % 2
            if j >= 1:
                descs_o[j - 1].wait()
            descs_g[j + 1] = pltpu.async_copy(
                table_hbm.at[idx_v.at[j + 1]], bufs[nb], gsems[nb])
        descs_g[j].wait()
        descs_o[j] = pltpu.async_copy(
            bufs[b], out_hbm.at[pl.ds(base + j * _CHUNK, _CHUNK)], osems[b])
    descs_o[_NCHUNK - 2].wait()
    descs_o[_NCHUNK - 1].wait()


def _sc_gather(table, idx):
    mesh = plsc.VectorSubcoreMesh(core_axis_name="c", subcore_axis_name="s")
    k = functools.partial(
        pl.kernel,
        mesh=mesh,
        out_type=jax.ShapeDtypeStruct((T, DP), jnp.float32),
        scratch_types=[
            pltpu.VMEM((_NCHUNK, _CHUNK), jnp.int32),
            pltpu.VMEM((_CHUNK, DP), jnp.float32),
            pltpu.VMEM((_CHUNK, DP), jnp.float32),
            pltpu.SemaphoreType.DMA,
            pltpu.SemaphoreType.DMA,
            pltpu.SemaphoreType.DMA,
            pltpu.SemaphoreType.DMA,
        ],
        compiler_params=pltpu.CompilerParams(use_tc_tiling_on_sc=True),
    )(_gather_body)
    return k(table, idx)


_BC = 4096


def _pad_body(tin_ref, out_ref):
    eye = jnp.eye(D, dtype=jnp.bfloat16)
    t = lax.dot_general(tin_ref[...].astype(jnp.bfloat16), eye,
                        (((0,), (0,)), ((), ())),
                        preferred_element_type=jnp.float32)
    out_ref[...] = jnp.concatenate([t, t], axis=1)


def _pad_table(tbl_t):
    return pl.pallas_call(
        _pad_body,
        grid=(pl.cdiv(V, _BC),),
        in_specs=[pl.BlockSpec((D, _BC), lambda i: (0, i))],
        out_specs=pl.BlockSpec((_BC, DP), lambda i: (i, 0)),
        out_shape=jax.ShapeDtypeStruct((V, DP), jnp.float32),
    )(tbl_t)


_BT = 2048
_GRID = T // _BT


def _morph_body(bnd_ref, emb_ref, m_ref, out_ref):
    g = pl.program_id(0)
    v = emb_ref[...]
    eye = jnp.eye(DP, dtype=jnp.float32)
    vt = lax.dot_general(eye, v, (((1,), (1,)), ((), ())),
                         preferred_element_type=jnp.float32)
    e_t = vt[:D, :]
    e_bf = e_t.astype(jnp.bfloat16)
    tok = g * _BT + lax.broadcasted_iota(jnp.int32, (1, _BT), 1)
    parts = []
    for u in range(U):
        m_u = jnp.logical_and(tok >= bnd_ref[u], tok < bnd_ref[u + 1])
        parts.append(e_bf * m_u.astype(jnp.bfloat16))
    b_t = jnp.concatenate(parts, axis=0)
    morph_t = lax.dot_general(m_ref[...], b_t, (((1,), (0,)), ((), ())),
                              preferred_element_type=jnp.float32)
    acc = e_t + morph_t
    n = jnp.sqrt(jnp.sum(acc * acc, axis=0, keepdims=True))
    out_ref[...] = acc / jnp.maximum(n, 1e-12)


def _tc_morph(bnd, emb128, m_t, interpret=False):
    grid_spec = pltpu.PrefetchScalarGridSpec(
        num_scalar_prefetch=1,
        grid=(_GRID,),
        in_specs=[
            pl.BlockSpec((_BT, DP), lambda i, bnd: (i, 0)),
            pl.BlockSpec((D, U * D), lambda i, bnd: (0, 0)),
        ],
        out_specs=pl.BlockSpec((D, _BT), lambda i, bnd: (0, i)),
    )
    return pl.pallas_call(
        _morph_body,
        grid_spec=grid_spec,
        out_shape=jax.ShapeDtypeStruct((D, T), jnp.float32),
        interpret=interpret,
    )(bnd, emb128, m_t)


def kernel(novel_items, novel_userids, item_emb, seg2_out):
    idx = novel_items.astype(jnp.int32).reshape(_NW, _NCHUNK, _CHUNK)
    tbl = _pad_table(jnp.transpose(item_emb))
    emb128 = _sc_gather(tbl, idx)
    m_t = jnp.transpose(seg2_out, (2, 0, 1)).reshape(D, U * D).astype(jnp.bfloat16)
    bnd = jnp.searchsorted(
        novel_userids, jnp.arange(U + 1, dtype=novel_userids.dtype)
    ).astype(jnp.int32)
    return jnp.transpose(_tc_morph(bnd, emb128, m_t))


# speedup vs baseline: 2.0158x; 1.1292x over previous
import functools

import jax
import jax.numpy as jnp
from jax import lax
from jax.experimental import pallas as pl
from jax.experimental.pallas import tpu as pltpu
from jax.experimental.pallas import tpu_sc as plsc

T = 32768
V = 100000
D = 64
U = 16
DP = 128

_NC = 2
_NS = 16
_NW = _NC * _NS
_ROWS_PER_W = T // _NW
_CHUNK = 128
_NCHUNK = _ROWS_PER_W // _CHUNK


def _gather_body(table_hbm, idx_hbm, out_hbm, idx_v, rows_a, rows_b,
                 gs0, gs1, os0, os1):
    wid = lax.axis_index("s") * _NC + lax.axis_index("c")
    pltpu.sync_copy(idx_hbm.at[wid], idx_v)
    base = wid * _ROWS_PER_W
    bufs, gsems, osems = (rows_a, rows_b), (gs0, gs1), (os0, os1)
    descs_g = [None] * _NCHUNK
    descs_o = [None] * _NCHUNK
    descs_g[0] = pltpu.async_copy(table_hbm.at[idx_v.at[0]], bufs[0], gsems[0])
    for j in range(_NCHUNK):
        b = j % 2
        if j + 1 < _NCHUNK:
            nb = (j + 1) % 2
            if j >= 1:
                descs_o[j - 1].wait()
            descs_g[j + 1] = pltpu.async_copy(
                table_hbm.at[idx_v.at[j + 1]], bufs[nb], gsems[nb])
        descs_g[j].wait()
        descs_o[j] = pltpu.async_copy(
            bufs[b], out_hbm.at[pl.ds(base + j * _CHUNK, _CHUNK)], osems[b])
    descs_o[_NCHUNK - 2].wait()
    descs_o[_NCHUNK - 1].wait()


def _sc_gather(table, idx):
    mesh = plsc.VectorSubcoreMesh(core_axis_name="c", subcore_axis_name="s")
    k = functools.partial(
        pl.kernel,
        mesh=mesh,
        out_type=jax.ShapeDtypeStruct((T, DP), jnp.float32),
        scratch_types=[
            pltpu.VMEM((_NCHUNK, _CHUNK), jnp.int32),
            pltpu.VMEM((_CHUNK, DP), jnp.float32),
            pltpu.VMEM((_CHUNK, DP), jnp.float32),
            pltpu.SemaphoreType.DMA,
            pltpu.SemaphoreType.DMA,
            pltpu.SemaphoreType.DMA,
            pltpu.SemaphoreType.DMA,
        ],
        compiler_params=pltpu.CompilerParams(use_tc_tiling_on_sc=True),
    )(_gather_body)
    return k(table, idx)


_BC = 8192


def _pad_body(tin_ref, out_ref):
    eye = jnp.eye(D, dtype=jnp.bfloat16)
    t = lax.dot_general(tin_ref[...].astype(jnp.bfloat16), eye,
                        (((0,), (0,)), ((), ())),
                        preferred_element_type=jnp.float32)
    out_ref[...] = jnp.concatenate([t, t], axis=1)


def _pad_table(tbl_t):
    return pl.pallas_call(
        _pad_body,
        grid=(pl.cdiv(V, _BC),),
        in_specs=[pl.BlockSpec((D, _BC), lambda i: (0, i))],
        out_specs=pl.BlockSpec((_BC, DP), lambda i: (i, 0)),
        out_shape=jax.ShapeDtypeStruct((V, DP), jnp.float32),
    )(tbl_t)


_BT = 4096
_GRID = T // _BT


def _morph_body(bnd_ref, emb_ref, m_ref, out_ref):
    g = pl.program_id(0)
    v = emb_ref[...]
    eye = jnp.eye(DP, dtype=jnp.float32)
    vt = lax.dot_general(eye, v, (((1,), (1,)), ((), ())),
                         preferred_element_type=jnp.float32)
    e_t = vt[:D, :]
    e_bf = e_t.astype(jnp.bfloat16)
    tok = g * _BT + lax.broadcasted_iota(jnp.int32, (1, _BT), 1)
    parts = []
    for u in range(U):
        m_u = jnp.logical_and(tok >= bnd_ref[u], tok < bnd_ref[u + 1])
        parts.append(e_bf * m_u.astype(jnp.bfloat16))
    b_t = jnp.concatenate(parts, axis=0)
    morph_t = lax.dot_general(m_ref[...], b_t, (((1,), (0,)), ((), ())),
                              preferred_element_type=jnp.float32)
    acc = e_t + morph_t
    n = jnp.sqrt(jnp.sum(acc * acc, axis=0, keepdims=True))
    out_ref[...] = acc / jnp.maximum(n, 1e-12)


def _tc_morph(bnd, emb128, m_t, interpret=False):
    grid_spec = pltpu.PrefetchScalarGridSpec(
        num_scalar_prefetch=1,
        grid=(_GRID,),
        in_specs=[
            pl.BlockSpec((_BT, DP), lambda i, bnd: (i, 0)),
            pl.BlockSpec((D, U * D), lambda i, bnd: (0, 0)),
        ],
        out_specs=pl.BlockSpec((D, _BT), lambda i, bnd: (0, i)),
    )
    return pl.pallas_call(
        _morph_body,
        grid_spec=grid_spec,
        out_shape=jax.ShapeDtypeStruct((D, T), jnp.float32),
        interpret=interpret,
    )(bnd, emb128, m_t)


def kernel(novel_items, novel_userids, item_emb, seg2_out):
    idx = novel_items.astype(jnp.int32).reshape(_NW, _NCHUNK, _CHUNK)
    tbl = _pad_table(jnp.transpose(item_emb))
    emb128 = _sc_gather(tbl, idx)
    m_t = jnp.transpose(seg2_out, (2, 0, 1)).reshape(D, U * D).astype(jnp.bfloat16)
    bnd = jnp.searchsorted(
        novel_userids, jnp.arange(U + 1, dtype=novel_userids.dtype)
    ).astype(jnp.int32)
    return jnp.transpose(_tc_morph(bnd, emb128, m_t))


# speedup vs baseline: 2.0601x; 1.0220x over previous
import functools

import jax
import jax.numpy as jnp
from jax import lax
from jax.experimental import pallas as pl
from jax.experimental.pallas import tpu as pltpu
from jax.experimental.pallas import tpu_sc as plsc

T = 32768
V = 100000
D = 64
U = 16
DP = 128

_NC = 2
_NS = 16
_NW = _NC * _NS
_ROWS_PER_W = T // _NW
_CHUNK = 128
_NCHUNK = _ROWS_PER_W // _CHUNK


def _gather_body(table_hbm, idx_hbm, out_hbm, idx_v, rows_a, rows_b,
                 gs0, gs1, os0, os1):
    wid = lax.axis_index("s") * _NC + lax.axis_index("c")
    pltpu.sync_copy(idx_hbm.at[wid], idx_v)
    base = wid * _ROWS_PER_W
    bufs, gsems, osems = (rows_a, rows_b), (gs0, gs1), (os0, os1)
    descs_g = [None] * _NCHUNK
    descs_o = [None] * _NCHUNK
    descs_g[0] = pltpu.async_copy(table_hbm.at[idx_v.at[0]], bufs[0], gsems[0])
    for j in range(_NCHUNK):
        b = j % 2
        if j + 1 < _NCHUNK:
            nb = (j + 1) % 2
            if j >= 1:
                descs_o[j - 1].wait()
            descs_g[j + 1] = pltpu.async_copy(
                table_hbm.at[idx_v.at[j + 1]], bufs[nb], gsems[nb])
        descs_g[j].wait()
        descs_o[j] = pltpu.async_copy(
            bufs[b], out_hbm.at[pl.ds(base + j * _CHUNK, _CHUNK)], osems[b])
    descs_o[_NCHUNK - 2].wait()
    descs_o[_NCHUNK - 1].wait()


def _sc_gather(table, idx):
    mesh = plsc.VectorSubcoreMesh(core_axis_name="c", subcore_axis_name="s")
    k = functools.partial(
        pl.kernel,
        mesh=mesh,
        out_type=jax.ShapeDtypeStruct((T, DP), jnp.float32),
        scratch_types=[
            pltpu.VMEM((_NCHUNK, _CHUNK), jnp.int32),
            pltpu.VMEM((_CHUNK, DP), jnp.float32),
            pltpu.VMEM((_CHUNK, DP), jnp.float32),
            pltpu.SemaphoreType.DMA,
            pltpu.SemaphoreType.DMA,
            pltpu.SemaphoreType.DMA,
            pltpu.SemaphoreType.DMA,
        ],
        compiler_params=pltpu.CompilerParams(use_tc_tiling_on_sc=True),
    )(_gather_body)
    return k(table, idx)


_BC = 16384


def _pad_body(tin_ref, out_ref):
    eye = jnp.eye(D, dtype=jnp.bfloat16)
    t = lax.dot_general(tin_ref[...].astype(jnp.bfloat16), eye,
                        (((0,), (0,)), ((), ())),
                        preferred_element_type=jnp.float32)
    out_ref[...] = jnp.concatenate([t, t], axis=1)


def _pad_table(tbl_t):
    return pl.pallas_call(
        _pad_body,
        grid=(pl.cdiv(V, _BC),),
        in_specs=[pl.BlockSpec((D, _BC), lambda i: (0, i))],
        out_specs=pl.BlockSpec((_BC, DP), lambda i: (i, 0)),
        out_shape=jax.ShapeDtypeStruct((V, DP), jnp.float32),
    )(tbl_t)


_BT = 8192
_GRID = T // _BT


def _morph_body(bnd_ref, emb_ref, m_ref, out_ref):
    g = pl.program_id(0)
    v = emb_ref[...]
    eye = jnp.eye(DP, dtype=jnp.float32)
    vt = lax.dot_general(eye, v, (((1,), (1,)), ((), ())),
                         preferred_element_type=jnp.float32)
    e_t = vt[:D, :]
    e_bf = e_t.astype(jnp.bfloat16)
    tok = g * _BT + lax.broadcasted_iota(jnp.int32, (1, _BT), 1)
    parts = []
    for u in range(U):
        m_u = jnp.logical_and(tok >= bnd_ref[u], tok < bnd_ref[u + 1])
        parts.append(e_bf * m_u.astype(jnp.bfloat16))
    b_t = jnp.concatenate(parts, axis=0)
    morph_t = lax.dot_general(m_ref[...], b_t, (((1,), (0,)), ((), ())),
                              preferred_element_type=jnp.float32)
    acc = e_t + morph_t
    n = jnp.sqrt(jnp.sum(acc * acc, axis=0, keepdims=True))
    out_ref[...] = acc / jnp.maximum(n, 1e-12)


def _tc_morph(bnd, emb128, m_t, interpret=False):
    grid_spec = pltpu.PrefetchScalarGridSpec(
        num_scalar_prefetch=1,
        grid=(_GRID,),
        in_specs=[
            pl.BlockSpec((_BT, DP), lambda i, bnd: (i, 0)),
            pl.BlockSpec((D, U * D), lambda i, bnd: (0, 0)),
        ],
        out_specs=pl.BlockSpec((D, _BT), lambda i, bnd: (0, i)),
    )
    return pl.pallas_call(
        _morph_body,
        grid_spec=grid_spec,
        out_shape=jax.ShapeDtypeStruct((D, T), jnp.float32),
        interpret=interpret,
    )(bnd, emb128, m_t)


def kernel(novel_items, novel_userids, item_emb, seg2_out):
    idx = novel_items.astype(jnp.int32).reshape(_NW, _NCHUNK, _CHUNK)
    tbl = _pad_table(jnp.transpose(item_emb))
    emb128 = _sc_gather(tbl, idx)
    m_t = jnp.transpose(seg2_out, (2, 0, 1)).reshape(D, U * D).astype(jnp.bfloat16)
    bnd = jnp.searchsorted(
        novel_userids, jnp.arange(U + 1, dtype=novel_userids.dtype)
    ).astype(jnp.int32)
    return jnp.transpose(_tc_morph(bnd, emb128, m_t))


# speedup vs baseline: 2.0628x; 1.0013x over previous
import functools

import jax
import jax.numpy as jnp
from jax import lax
from jax.experimental import pallas as pl
from jax.experimental.pallas import tpu as pltpu
from jax.experimental.pallas import tpu_sc as plsc

T = 32768
V = 100000
D = 64
U = 16
DP = 128

_NC = 2
_NS = 16
_NW = _NC * _NS
_ROWS_PER_W = T // _NW
_CHUNK = 128
_NCHUNK = _ROWS_PER_W // _CHUNK


def _gather_body(table_hbm, idx_hbm, out_hbm, idx_v, rows_a, rows_b,
                 gs0, gs1, os0, os1):
    wid = lax.axis_index("s") * _NC + lax.axis_index("c")
    pltpu.sync_copy(idx_hbm.at[wid], idx_v)
    base = wid * _ROWS_PER_W
    bufs, gsems, osems = (rows_a, rows_b), (gs0, gs1), (os0, os1)
    descs_g = [None] * _NCHUNK
    descs_o = [None] * _NCHUNK
    descs_g[0] = pltpu.async_copy(table_hbm.at[idx_v.at[0]], bufs[0], gsems[0])
    for j in range(_NCHUNK):
        b = j % 2
        if j + 1 < _NCHUNK:
            nb = (j + 1) % 2
            if j >= 1:
                descs_o[j - 1].wait()
            descs_g[j + 1] = pltpu.async_copy(
                table_hbm.at[idx_v.at[j + 1]], bufs[nb], gsems[nb])
        descs_g[j].wait()
        descs_o[j] = pltpu.async_copy(
            bufs[b], out_hbm.at[pl.ds(base + j * _CHUNK, _CHUNK)], osems[b])
    descs_o[_NCHUNK - 2].wait()
    descs_o[_NCHUNK - 1].wait()


def _sc_gather(table, idx):
    mesh = plsc.VectorSubcoreMesh(core_axis_name="c", subcore_axis_name="s")
    k = functools.partial(
        pl.kernel,
        mesh=mesh,
        out_type=jax.ShapeDtypeStruct((T, DP), jnp.float32),
        scratch_types=[
            pltpu.VMEM((_NCHUNK, _CHUNK), jnp.int32),
            pltpu.VMEM((_CHUNK, DP), jnp.float32),
            pltpu.VMEM((_CHUNK, DP), jnp.float32),
            pltpu.SemaphoreType.DMA,
            pltpu.SemaphoreType.DMA,
            pltpu.SemaphoreType.DMA,
            pltpu.SemaphoreType.DMA,
        ],
        compiler_params=pltpu.CompilerParams(use_tc_tiling_on_sc=True),
    )(_gather_body)
    return k(table, idx)


_BC = 16384


def _pad_body(tin_ref, out_ref):
    eye = jnp.eye(D, dtype=jnp.bfloat16)
    t = lax.dot_general(tin_ref[...].astype(jnp.bfloat16), eye,
                        (((0,), (0,)), ((), ())),
                        preferred_element_type=jnp.float32)
    out_ref[...] = jnp.concatenate([t, t], axis=1)


def _pad_table(tbl_t):
    return pl.pallas_call(
        _pad_body,
        grid=(pl.cdiv(V, _BC),),
        in_specs=[pl.BlockSpec((D, _BC), lambda i: (0, i))],
        out_specs=pl.BlockSpec((_BC, DP), lambda i: (i, 0)),
        out_shape=jax.ShapeDtypeStruct((V, DP), jnp.float32),
    )(tbl_t)


_BT = 8192
_GRID = T // _BT


def _morph_body(bnd_ref, emb_ref, m_ref, out_ref):
    g = pl.program_id(0)
    v = emb_ref[...].astype(jnp.bfloat16)
    eye = jnp.eye(DP, dtype=jnp.bfloat16)
    vt = lax.dot_general(eye, v, (((1,), (1,)), ((), ())),
                         preferred_element_type=jnp.float32)
    e_t = vt[:D, :]
    e_bf = e_t.astype(jnp.bfloat16)
    tok = g * _BT + lax.broadcasted_iota(jnp.int32, (1, _BT), 1)
    parts = []
    for u in range(U):
        m_u = jnp.logical_and(tok >= bnd_ref[u], tok < bnd_ref[u + 1])
        parts.append(e_bf * m_u.astype(jnp.bfloat16))
    b_t = jnp.concatenate(parts, axis=0)
    morph_t = lax.dot_general(m_ref[...], b_t, (((1,), (0,)), ((), ())),
                              preferred_element_type=jnp.float32)
    acc = e_t + morph_t
    n = jnp.sqrt(jnp.sum(acc * acc, axis=0, keepdims=True))
    out_ref[...] = acc / jnp.maximum(n, 1e-12)


def _tc_morph(bnd, emb128, m_t, interpret=False):
    grid_spec = pltpu.PrefetchScalarGridSpec(
        num_scalar_prefetch=1,
        grid=(_GRID,),
        in_specs=[
            pl.BlockSpec((_BT, DP), lambda i, bnd: (i, 0)),
            pl.BlockSpec((D, U * D), lambda i, bnd: (0, 0)),
        ],
        out_specs=pl.BlockSpec((D, _BT), lambda i, bnd: (0, i)),
    )
    return pl.pallas_call(
        _morph_body,
        grid_spec=grid_spec,
        out_shape=jax.ShapeDtypeStruct((D, T), jnp.float32),
        interpret=interpret,
    )(bnd, emb128, m_t)


def kernel(novel_items, novel_userids, item_emb, seg2_out):
    idx = novel_items.astype(jnp.int32).reshape(_NW, _NCHUNK, _CHUNK)
    tbl = _pad_table(jnp.transpose(item_emb))
    emb128 = _sc_gather(tbl, idx)
    m_t = jnp.transpose(seg2_out, (2, 0, 1)).reshape(D, U * D).astype(jnp.bfloat16)
    bnd = jnp.searchsorted(
        novel_userids, jnp.arange(U + 1, dtype=novel_userids.dtype)
    ).astype(jnp.int32)
    return jnp.transpose(_tc_morph(bnd, emb128, m_t))
